# Initial kernel scaffold; baseline (speedup 1.0000x reference)
import functools

import jax
import jax.numpy as jnp
from jax import lax
from jax.experimental import pallas as pl
from jax.experimental.pallas import tpu as pltpu
from jax.experimental.pallas import tpu_sc as plsc

_N = 10000
_INC = 320000
_D = 128
_G = 64
_NCLS = 10
_NC, _NS = 2, 16
_CHUNK = 128
_NCHUNKS = _INC // _CHUNK
_CPW = -(-_NCHUNKS // (_NC * _NS))
_RPT = _N // _NS


def _counts_body(nidx, eidx, out_n, out_e, cnt_sh, ones_v, zbuf, idxv):
    cid = lax.axis_index("c")
    sid = lax.axis_index("s")
    one16 = jnp.ones((16,), jnp.float32)
    zero16 = jnp.zeros((16,), jnp.float32)

    def fill(r, _):
        ones_v[r, :] = one16
        zbuf[r, :] = zero16
        return _

    lax.fori_loop(0, 125, fill, None)

    def fill2(r, _):
        ones_v[r, :] = one16
        return _

    lax.fori_loop(125, 128, fill2, None)

    for k in range(5):
        pltpu.sync_copy(zbuf, cnt_sh.at[pl.ds(sid * _RPT + k * 125, 125)])
    plsc.subcore_barrier()

    def chunk(j, _):
        c = sid + _NS * j

        @pl.when(c < _NCHUNKS)
        def _do():
            @pl.when(cid == 0)
            def _n():
                pltpu.sync_copy(nidx.at[c], idxv)

            @pl.when(cid == 1)
            def _e():
                pltpu.sync_copy(eidx.at[c], idxv)

            pltpu.sync_copy(ones_v, cnt_sh.at[idxv], add=True)

        return _

    lax.fori_loop(0, -(-_NCHUNKS // _NS), chunk, None)
    plsc.subcore_barrier()

    @pl.when(cid == 0)
    def _wn():
        pltpu.sync_copy(cnt_sh.at[pl.ds(sid * _RPT, _RPT)],
                        out_n.at[pl.ds(sid * _RPT, _RPT)])

    @pl.when(cid == 1)
    def _we():
        pltpu.sync_copy(cnt_sh.at[pl.ds(sid * _RPT, _RPT)],
                        out_e.at[pl.ds(sid * _RPT, _RPT)])


_counts_kernel = pl.kernel(
    _counts_body,
    out_type=(jax.ShapeDtypeStruct((_N, 16), jnp.float32),
              jax.ShapeDtypeStruct((_N, 16), jnp.float32)),
    mesh=plsc.VectorSubcoreMesh(core_axis_name="c", subcore_axis_name="s"),
    scratch_types=[
        pltpu.VMEM_SHARED((_N, 16), jnp.float32),
        pltpu.VMEM((128, 16), jnp.float32),
        pltpu.VMEM((125, 16), jnp.float32),
        pltpu.VMEM((_CHUNK,), jnp.int32),
    ],
)


def _scatter_body(table, src_idx, dst_idx, out, idx_s, idx_d, rows, acc, sem):
    cid = lax.axis_index("c")
    sid = lax.axis_index("s")
    wid = sid * _NC + cid
    zero16 = jnp.zeros((16,), jnp.float32)

    def zrow(r, _):
        for k in range(8):
            rows[r, pl.ds(k * 16, 16)] = zero16
        return _

    lax.fori_loop(0, 125, zrow, None)
    for k in range(5):
        pltpu.sync_copy(rows.at[pl.ds(0, 125)],
                        acc.at[pl.ds(sid * _RPT + k * 125, 125)])
    plsc.subcore_barrier()

    def chunk(j, _):
        c = wid + _NC * _NS * j

        @pl.when(c < _NCHUNKS)
        def _do():
            pltpu.sync_copy(src_idx.at[c], idx_s)
            pltpu.sync_copy(dst_idx.at[c], idx_d)
            pltpu.async_copy(table.at[idx_s], rows, sem).wait()
            pltpu.sync_copy(rows, acc.at[idx_d], add=True)

        return _

    lax.fori_loop(0, _CPW, chunk, None)
    plsc.subcore_barrier()
    pltpu.sync_copy(acc.at[pl.ds(sid * _RPT, _RPT)],
                    out.at[cid, pl.ds(sid * _RPT, _RPT)])


_scatter_kernel = pl.kernel(
    _scatter_body,
    out_type=jax.ShapeDtypeStruct((_NC, _N, _D), jnp.float32),
    mesh=plsc.VectorSubcoreMesh(core_axis_name="c", subcore_axis_name="s"),
    scratch_types=[
        pltpu.VMEM((_CHUNK,), jnp.int32),
        pltpu.VMEM((_CHUNK,), jnp.int32),
        pltpu.VMEM((_CHUNK, _D), jnp.float32),
        pltpu.VMEM_SHARED((_N, _D), jnp.float32),
        pltpu.SemaphoreType.DMA,
    ],
)


_RB = 500
_NRB = _N // _RB


def _mm_body(x_ref, w_ref, o_ref):
    o_ref[...] = jnp.dot(x_ref[...], w_ref[...],
                         preferred_element_type=jnp.float32)


_mm = pl.pallas_call(
    _mm_body,
    grid=(_NRB,),
    in_specs=[pl.BlockSpec((_RB, _D), lambda i: (i, 0)),
              pl.BlockSpec((_D, _D), lambda i: (0, 0))],
    out_specs=pl.BlockSpec((_RB, _D), lambda i: (i, 0)),
    out_shape=jax.ShapeDtypeStruct((_N, _D), jnp.float32),
)


def _inv(c):
    return jnp.where(c == 0.0, 0.0, 1.0 / c)


def _ecomb_body(p_ref, c_ref, o_ref):
    o_ref[...] = _inv(c_ref[...]) * (p_ref[0] + p_ref[1])


_ecomb = pl.pallas_call(
    _ecomb_body,
    grid=(_NRB,),
    in_specs=[pl.BlockSpec((_NC, _RB, _D), lambda i: (0, i, 0)),
              pl.BlockSpec((_RB, 1), lambda i: (i, 0))],
    out_specs=pl.BlockSpec((_RB, _D), lambda i: (i, 0)),
    out_shape=jax.ShapeDtypeStruct((_N, _D), jnp.float32),
)


def _vcomb_mm_body(q_ref, c_ref, b_ref, w_ref, o_ref):
    v = _inv(c_ref[...]) * (q_ref[0] + q_ref[1]) + b_ref[...]
    v = jnp.maximum(v, 0.0)
    o_ref[...] = jnp.dot(v, w_ref[...], preferred_element_type=jnp.float32)


_vcomb_mm = pl.pallas_call(
    _vcomb_mm_body,
    grid=(_NRB,),
    in_specs=[pl.BlockSpec((_NC, _RB, _D), lambda i: (0, i, 0)),
              pl.BlockSpec((_RB, 1), lambda i: (i, 0)),
              pl.BlockSpec((1, _D), lambda i: (0, 0)),
              pl.BlockSpec((_D, _D), lambda i: (0, 0))],
    out_specs=pl.BlockSpec((_RB, _D), lambda i: (i, 0)),
    out_shape=jax.ShapeDtypeStruct((_N, _D), jnp.float32),
)


def _final_body(q_ref, c_ref, b_ref, bat_ref, wl_ref, bl_ref, o_ref,
                sums, cnts):
    i = pl.program_id(0)

    @pl.when(i == 0)
    def _init():
        sums[...] = jnp.zeros_like(sums)
        cnts[...] = jnp.zeros_like(cnts)

    v = _inv(c_ref[...]) * (q_ref[0] + q_ref[1]) + b_ref[...]
    h = jnp.maximum(v, 0.0)
    gids = lax.broadcasted_iota(jnp.float32, (_G, _RB), 0)
    onehot = (bat_ref[...] == gids).astype(jnp.float32)
    sums[...] += jnp.dot(onehot, h, preferred_element_type=jnp.float32)
    cnts[...] += jnp.sum(onehot, axis=1, keepdims=True)

    @pl.when(i == _NRB - 1)
    def _fin():
        g = sums[...] / jnp.maximum(cnts[...], 1.0)
        o_ref[...] = (jnp.dot(g, wl_ref[...],
                              preferred_element_type=jnp.float32)
                      + bl_ref[...])


_final = pl.pallas_call(
    _final_body,
    grid=(_NRB,),
    in_specs=[pl.BlockSpec((_NC, _RB, _D), lambda i: (0, i, 0)),
              pl.BlockSpec((_RB, 1), lambda i: (i, 0)),
              pl.BlockSpec((1, _D), lambda i: (0, 0)),
              pl.BlockSpec((1, _RB), lambda i: (0, i)),
              pl.BlockSpec((_D, _NCLS), lambda i: (0, 0)),
              pl.BlockSpec((1, _NCLS), lambda i: (0, 0))],
    out_specs=pl.BlockSpec((_G, _NCLS), lambda i: (0, 0)),
    out_shape=jax.ShapeDtypeStruct((_G, _NCLS), jnp.float32),
    scratch_shapes=[pltpu.VMEM((_G, _D), jnp.float32),
                    pltpu.VMEM((_G, 1), jnp.float32)],
)


def kernel(x, hyperedge_index, batch, W0, b0, W1, b1, Wlin, blin):
    node2d = hyperedge_index[0].reshape(_NCHUNKS, _CHUNK)
    edge2d = hyperedge_index[1].reshape(_NCHUNKS, _CHUNK)

    cnt_n, cnt_e = _counts_kernel(node2d, edge2d)
    c_n = cnt_n[:, :1]
    c_e = cnt_e[:, :1]

    h0 = _mm(x, W0)
    p1 = _scatter_kernel(h0, node2d, edge2d)
    e1 = _ecomb(p1, c_e)
    q1 = _scatter_kernel(e1, edge2d, node2d)
    h1 = _vcomb_mm(q1, c_n, b0.reshape(1, _D), W1)
    p2 = _scatter_kernel(h1, node2d, edge2d)
    e2 = _ecomb(p2, c_e)
    q2 = _scatter_kernel(e2, edge2d, node2d)
    out = _final(q2, c_n, b1.reshape(1, _D),
                 batch.astype(jnp.float32).reshape(1, _N),
                 Wlin, blin.reshape(1, _NCLS))
    return out


# speedup vs baseline: 11.2622x; 11.2622x over previous
import functools

import jax
import jax.numpy as jnp
from jax import lax
from jax.experimental import pallas as pl
from jax.experimental.pallas import tpu as pltpu
from jax.experimental.pallas import tpu_sc as plsc

_N = 10000
_INC = 320000
_D = 128
_G = 64
_NCLS = 10
_NC, _NS = 2, 16
_CHUNK = 128
_NCHUNKS = _INC // _CHUNK
_CPW = -(-_NCHUNKS // (_NC * _NS))
_NP = 10240
_RPT = _NP // _NS


def _counts_body(hei, out_n, out_e, idxv, cnt, iotav, acc_sh):
    cid = lax.axis_index("c")
    sid = lax.axis_index("s")
    zero16 = jnp.zeros((16,), jnp.float32)
    ones16 = jnp.ones((16,), jnp.float32)

    def zrow(r, _):
        for k in range(8):
            cnt[r, pl.ds(k * 16, 16)] = zero16
        return _

    lax.fori_loop(0, 128, zrow, None)
    for k in range(8):
        iotav[pl.ds(k * 16, 16)] = lax.iota(jnp.int32, 16) + (16 * k)
    pltpu.sync_copy(cnt.at[pl.ds(0, 8)], acc_sh.at[pl.ds(sid * 8, 8)])
    plsc.subcore_barrier()

    def histo(row):
        def chunk(j, _):
            c = sid + _NS * j

            @pl.when(c < _NCHUNKS)
            def _do():
                pltpu.sync_copy(hei.at[row, pl.ds(c * _CHUNK, _CHUNK)], idxv)
                for k in range(8):
                    iv = idxv[pl.ds(k * 16, 16)]
                    plsc.addupdate_scatter(
                        cnt,
                        [lax.shift_right_logical(iv, 7),
                         jnp.bitwise_and(iv, 127)],
                        ones16)

            return _

        lax.fori_loop(0, -(-_NCHUNKS // _NS), chunk, None)

    @pl.when(cid == 0)
    def _hn():
        histo(0)

    @pl.when(cid == 1)
    def _he():
        histo(1)

    pltpu.sync_copy(cnt, acc_sh.at[iotav], add=True)
    plsc.subcore_barrier()

    @pl.when(cid == 0)
    def _wn():
        pltpu.sync_copy(acc_sh.at[pl.ds(sid * 8, 8)], out_n.at[pl.ds(sid * 8, 8)])

    @pl.when(cid == 1)
    def _we():
        pltpu.sync_copy(acc_sh.at[pl.ds(sid * 8, 8)], out_e.at[pl.ds(sid * 8, 8)])


_counts_kernel = pl.kernel(
    _counts_body,
    out_type=(jax.ShapeDtypeStruct((128, 128), jnp.float32),
              jax.ShapeDtypeStruct((128, 128), jnp.float32)),
    mesh=plsc.VectorSubcoreMesh(core_axis_name="c", subcore_axis_name="s"),
    compiler_params=pltpu.CompilerParams(needs_layout_passes=False),
    scratch_types=[
        pltpu.VMEM((_CHUNK,), jnp.int32),
        pltpu.VMEM((128, 128), jnp.float32),
        pltpu.VMEM((128,), jnp.int32),
        pltpu.VMEM_SHARED((128, 128), jnp.float32),
    ],
)


def _scatter_body(src_row, dst_row, table, hei, out, idx_s, idx_d, rows, acc, sem):
    cid = lax.axis_index("c")
    sid = lax.axis_index("s")
    wid = sid * _NC + cid
    zero16 = jnp.zeros((16,), jnp.float32)

    def zrow(r, _):
        for k in range(8):
            rows[r, pl.ds(k * 16, 16)] = zero16
        return _

    lax.fori_loop(0, 128, zrow, None)
    for k in range(5):
        pltpu.sync_copy(rows, acc.at[pl.ds(sid * _RPT + k * 128, 128)])
    plsc.subcore_barrier()

    def chunk(j, _):
        c = wid + _NC * _NS * j

        @pl.when(c < _NCHUNKS)
        def _do():
            pltpu.sync_copy(hei.at[src_row, pl.ds(c * _CHUNK, _CHUNK)], idx_s)
            pltpu.sync_copy(hei.at[dst_row, pl.ds(c * _CHUNK, _CHUNK)], idx_d)
            pltpu.async_copy(table.at[idx_s], rows, sem).wait()
            pltpu.sync_copy(rows, acc.at[idx_d], add=True)

        return _

    lax.fori_loop(0, _CPW, chunk, None)
    plsc.subcore_barrier()
    pltpu.sync_copy(acc.at[pl.ds(sid * _RPT, _RPT)],
                    out.at[cid, pl.ds(sid * _RPT, _RPT)])


def _make_scatter(src_row, dst_row):
    return pl.kernel(
        functools.partial(_scatter_body, src_row, dst_row),
        out_type=jax.ShapeDtypeStruct((_NC, _NP, _D), jnp.float32),
        mesh=plsc.VectorSubcoreMesh(core_axis_name="c", subcore_axis_name="s"),
        compiler_params=pltpu.CompilerParams(needs_layout_passes=False),
        scratch_types=[
            pltpu.VMEM((_CHUNK,), jnp.int32),
            pltpu.VMEM((_CHUNK,), jnp.int32),
            pltpu.VMEM((_CHUNK, _D), jnp.float32),
            pltpu.VMEM_SHARED((_NP, _D), jnp.float32),
            pltpu.SemaphoreType.DMA,
        ],
    )


_scatter_n2e = _make_scatter(0, 1)
_scatter_e2n = _make_scatter(1, 0)


_RB = 1000
_NRB = _N // _RB


def _mm_body(x_ref, w_ref, o_ref):
    o_ref[...] = jnp.dot(x_ref[...], w_ref[...],
                         preferred_element_type=jnp.float32)


_mm = pl.pallas_call(
    _mm_body,
    grid=(_NRB,),
    in_specs=[pl.BlockSpec((_RB, _D), lambda i: (i, 0)),
              pl.BlockSpec((_D, _D), lambda i: (0, 0))],
    out_specs=pl.BlockSpec((_RB, _D), lambda i: (i, 0)),
    out_shape=jax.ShapeDtypeStruct((_N, _D), jnp.float32),
)


def _inv(c):
    return jnp.where(c == 0.0, 0.0, 1.0 / c)


def _ecomb_body(p_ref, c_ref, o_ref):
    o_ref[...] = _inv(c_ref[...]) * (p_ref[0] + p_ref[1])


_ecomb = pl.pallas_call(
    _ecomb_body,
    grid=(_NRB,),
    in_specs=[pl.BlockSpec((_NC, _RB, _D), lambda i: (0, i, 0)),
              pl.BlockSpec((_RB, 1), lambda i: (i, 0))],
    out_specs=pl.BlockSpec((_RB, _D), lambda i: (i, 0)),
    out_shape=jax.ShapeDtypeStruct((_N, _D), jnp.float32),
)


def _vcomb_mm_body(q_ref, c_ref, b_ref, w_ref, o_ref):
    v = _inv(c_ref[...]) * (q_ref[0] + q_ref[1]) + b_ref[...]
    v = jnp.maximum(v, 0.0)
    o_ref[...] = jnp.dot(v, w_ref[...], preferred_element_type=jnp.float32)


_vcomb_mm = pl.pallas_call(
    _vcomb_mm_body,
    grid=(_NRB,),
    in_specs=[pl.BlockSpec((_NC, _RB, _D), lambda i: (0, i, 0)),
              pl.BlockSpec((_RB, 1), lambda i: (i, 0)),
              pl.BlockSpec((1, _D), lambda i: (0, 0)),
              pl.BlockSpec((_D, _D), lambda i: (0, 0))],
    out_specs=pl.BlockSpec((_RB, _D), lambda i: (i, 0)),
    out_shape=jax.ShapeDtypeStruct((_N, _D), jnp.float32),
)


def _final_body(q_ref, c_ref, b_ref, bat_ref, wl_ref, bl_ref, o_ref,
                sums, cnts):
    i = pl.program_id(0)

    @pl.when(i == 0)
    def _init():
        sums[...] = jnp.zeros_like(sums)
        cnts[...] = jnp.zeros_like(cnts)

    v = _inv(c_ref[...]) * (q_ref[0] + q_ref[1]) + b_ref[...]
    h = jnp.maximum(v, 0.0)
    gids = lax.broadcasted_iota(jnp.int32, (_G, _RB), 0).astype(jnp.float32)
    onehot = (bat_ref[0] == gids).astype(jnp.float32)
    sums[...] += jnp.dot(onehot, h, preferred_element_type=jnp.float32)
    cnts[...] = cnts[...] + jnp.sum(onehot, axis=1, keepdims=True)

    @pl.when(i == _NRB - 1)
    def _fin():
        g = sums[...] / jnp.maximum(cnts[...], 1.0)
        o_ref[...] = (jnp.dot(g, wl_ref[...],
                              preferred_element_type=jnp.float32)
                      + bl_ref[...])


_final = pl.pallas_call(
    _final_body,
    grid=(_NRB,),
    in_specs=[pl.BlockSpec((_NC, _RB, _D), lambda i: (0, i, 0)),
              pl.BlockSpec((_RB, 1), lambda i: (i, 0)),
              pl.BlockSpec((1, _D), lambda i: (0, 0)),
              pl.BlockSpec((1, 1, _RB), lambda i: (i, 0, 0)),
              pl.BlockSpec((_D, _NCLS), lambda i: (0, 0)),
              pl.BlockSpec((1, _NCLS), lambda i: (0, 0))],
    out_specs=pl.BlockSpec((_G, _NCLS), lambda i: (0, 0)),
    out_shape=jax.ShapeDtypeStruct((_G, _NCLS), jnp.float32),
    scratch_shapes=[pltpu.VMEM((_G, _D), jnp.float32),
                    pltpu.VMEM((_G, 128), jnp.float32)],
)


def kernel(x, hyperedge_index, batch, W0, b0, W1, b1, Wlin, blin):
    cnt_n, cnt_e = _counts_kernel(hyperedge_index)
    c_n = cnt_n.reshape(-1)[:_N].reshape(_N, 1)
    c_e = cnt_e.reshape(-1)[:_N].reshape(_N, 1)

    h0 = _mm(x, W0)
    p1 = _scatter_n2e(h0, hyperedge_index)
    e1 = _ecomb(p1, c_e)
    q1 = _scatter_e2n(e1, hyperedge_index)
    h1 = _vcomb_mm(q1, c_n, b0.reshape(1, _D), W1)
    p2 = _scatter_n2e(h1, hyperedge_index)
    e2 = _ecomb(p2, c_e)
    q2 = _scatter_e2n(e2, hyperedge_index)
    out = _final(q2, c_n, b1.reshape(1, _D),
                 batch.astype(jnp.float32).reshape(_NRB, 1, _RB),
                 Wlin, blin.reshape(1, _NCLS))
    return out
